# baseline (speedup 1.0000x reference)
import functools

import jax
import jax.numpy as jnp
from jax.experimental import pallas as pl
from jax.experimental.pallas import tpu as pltpu

NUM_EXPERTS = 64
TOP_K = 2
D_MODEL = 768
D_FF = 1024
SEQ = 2048
BM = 128
N_ASSIGN = SEQ * TOP_K
NUM_BLOCKS = N_ASSIGN // BM + NUM_EXPERTS


def _moe_block_kernel(
    be_ref, act_ref, x_ref, tok_col_ref, tok_row_ref, w_col_ref,
    wg_ref, wu_ref, wd_ref, out_ref
):
    i = pl.program_id(0)

    @pl.when(i == 0)
    def _():
        out_ref[...] = jnp.zeros_like(out_ref)

    @pl.when(act_ref[i] == 1)
    def _():
        ids_col = tok_col_ref[...]
        gmat = (
            jax.lax.broadcasted_iota(jnp.int32, (BM, SEQ), 1) == ids_col
        ).astype(jnp.bfloat16)
        xb = jnp.dot(gmat, x_ref[...], preferred_element_type=jnp.float32)

        g = jnp.dot(xb, wg_ref[0], preferred_element_type=jnp.float32)
        u = jnp.dot(xb, wu_ref[0], preferred_element_type=jnp.float32)
        h = (g * jax.nn.sigmoid(g)) * u
        y = jnp.dot(h, wd_ref[0], preferred_element_type=jnp.float32)

        ids_row = tok_row_ref[0, 0:1, :]
        w_col = w_col_ref[...]
        y_w = (y * w_col).astype(jnp.bfloat16)
        smat = (
            jax.lax.broadcasted_iota(jnp.int32, (SEQ, BM), 0) == ids_row
        ).astype(jnp.bfloat16)
        out_ref[...] += jnp.dot(smat, y_w, preferred_element_type=jnp.float32)


@jax.jit
def kernel(x, gate_w, Wg, Wu, Wd):
    x0 = x[0]

    scores = x0 @ gate_w
    eids = jnp.arange(NUM_EXPERTS, dtype=jnp.int32)
    i1 = jnp.argmax(scores, axis=-1).astype(jnp.int32)
    v1 = jnp.max(scores, axis=-1)
    masked = jnp.where(eids[None, :] == i1[:, None], -jnp.inf, scores)
    i2 = jnp.argmax(masked, axis=-1).astype(jnp.int32)
    v2 = jnp.max(masked, axis=-1)
    s1 = jax.nn.sigmoid(v1 - v2)
    tw = jnp.stack([s1, 1.0 - s1], axis=-1)

    e_flat = jnp.stack([i1, i2], axis=-1).reshape(-1)
    t_flat = jnp.arange(N_ASSIGN, dtype=jnp.int32) // TOP_K

    onehot = (e_flat[:, None] == eids[None, :]).astype(jnp.int32)
    csum = jnp.cumsum(onehot, axis=0)
    gs = csum[-1]
    rank = jnp.sum((csum - onehot) * onehot, axis=1)

    blocks_per = (gs + BM - 1) // BM
    bcum = jnp.cumsum(blocks_per)
    bstart = bcum - blocks_per
    n_active = bcum[-1]

    p = bstart[e_flat] * BM + rank
    tok_pad = jnp.full((NUM_BLOCKS * BM,), SEQ, jnp.int32).at[p].set(t_flat)
    w_pad = jnp.zeros((NUM_BLOCKS * BM,), jnp.float32).at[p].set(tw.reshape(-1))

    blk_ids = jnp.arange(NUM_BLOCKS, dtype=jnp.int32)
    be = jnp.searchsorted(
        bcum, jnp.minimum(blk_ids, n_active - 1), side="right"
    ).astype(jnp.int32)
    block_expert = jnp.minimum(be, NUM_EXPERTS - 1)
    active = (blk_ids < n_active).astype(jnp.int32)

    grid_spec = pltpu.PrefetchScalarGridSpec(
        num_scalar_prefetch=2,
        grid=(NUM_BLOCKS,),
        in_specs=[
            pl.BlockSpec((SEQ, D_MODEL), lambda i, be_r, a_r: (0, 0)),
            pl.BlockSpec((BM, 1), lambda i, be_r, a_r: (i, 0)),
            pl.BlockSpec((1, 8, BM), lambda i, be_r, a_r: (i, 0, 0)),
            pl.BlockSpec((BM, 1), lambda i, be_r, a_r: (i, 0)),
            pl.BlockSpec((1, D_MODEL, D_FF), lambda i, be_r, a_r: (be_r[i], 0, 0)),
            pl.BlockSpec((1, D_MODEL, D_FF), lambda i, be_r, a_r: (be_r[i], 0, 0)),
            pl.BlockSpec((1, D_FF, D_MODEL), lambda i, be_r, a_r: (be_r[i], 0, 0)),
        ],
        out_specs=pl.BlockSpec((SEQ, D_MODEL), lambda i, be_r, a_r: (0, 0)),
    )

    out = pl.pallas_call(
        _moe_block_kernel,
        grid_spec=grid_spec,
        out_shape=jax.ShapeDtypeStruct((SEQ, D_MODEL), jnp.float32),
    )(
        block_expert,
        active,
        x0.astype(jnp.bfloat16),
        tok_pad.reshape(NUM_BLOCKS * BM, 1),
        jnp.broadcast_to(tok_pad.reshape(NUM_BLOCKS, 1, BM), (NUM_BLOCKS, 8, BM)),
        w_pad.reshape(NUM_BLOCKS * BM, 1),
        Wg,
        Wu,
        Wd,
    )
    return out[None]


# speedup vs baseline: 1.2206x; 1.2206x over previous
import functools

import jax
import jax.numpy as jnp
from jax.experimental import pallas as pl
from jax.experimental.pallas import tpu as pltpu

NUM_EXPERTS = 64
TOP_K = 2
D_MODEL = 768
D_FF = 1024
SEQ = 2048
BM = 128
N_ASSIGN = SEQ * TOP_K
NUM_BLOCKS = N_ASSIGN // BM + NUM_EXPERTS


def _moe_block_kernel(
    be_ref, act_ref, x_ref, tok_col_ref, tok_row_ref, w_row_ref,
    wg_ref, wu_ref, wd_ref, out_ref
):
    i = pl.program_id(0)

    @pl.when(i == 0)
    def _():
        out_ref[...] = jnp.zeros_like(out_ref)

    @pl.when(act_ref[i] == 1)
    def _():
        ids_col = tok_col_ref[...]
        gmat = (
            jax.lax.broadcasted_iota(jnp.int32, (BM, SEQ), 1) == ids_col
        ).astype(jnp.float32)
        xb = jnp.dot(gmat, x_ref[...], preferred_element_type=jnp.float32)

        g = jnp.dot(xb, wg_ref[0], preferred_element_type=jnp.float32)
        u = jnp.dot(xb, wu_ref[0], preferred_element_type=jnp.float32)
        h = (g * jax.nn.sigmoid(g)) * u
        y = jnp.dot(h, wd_ref[0], preferred_element_type=jnp.float32)

        ids_row = tok_row_ref[0, 0:1, :]
        w_row = w_row_ref[0, 0:1, :]
        smat = (
            jax.lax.broadcasted_iota(jnp.int32, (SEQ, BM), 0) == ids_row
        ).astype(jnp.float32) * w_row
        out_ref[...] += jnp.dot(smat, y, preferred_element_type=jnp.float32)


@jax.jit
def kernel(x, gate_w, Wg, Wu, Wd):
    x0 = x[0]

    scores = x0 @ gate_w
    eids = jnp.arange(NUM_EXPERTS, dtype=jnp.int32)
    i1 = jnp.argmax(scores, axis=-1).astype(jnp.int32)
    v1 = jnp.max(scores, axis=-1)
    masked = jnp.where(eids[None, :] == i1[:, None], -jnp.inf, scores)
    i2 = jnp.argmax(masked, axis=-1).astype(jnp.int32)
    v2 = jnp.max(masked, axis=-1)
    s1 = jax.nn.sigmoid(v1 - v2)
    tw = jnp.stack([s1, 1.0 - s1], axis=-1)

    e_flat = jnp.stack([i1, i2], axis=-1).reshape(-1)
    t_flat = jnp.repeat(jnp.arange(SEQ, dtype=jnp.int32), TOP_K)

    onehot = (e_flat[:, None] == eids[None, :]).astype(jnp.int32)
    csum = jnp.cumsum(onehot, axis=0)
    gs = csum[-1]
    rank = jnp.sum((csum - onehot) * onehot, axis=1)

    blocks_per = (gs + BM - 1) // BM
    bcum = jnp.cumsum(blocks_per)
    bstart = bcum - blocks_per
    n_active = bcum[-1]

    bstart_g = (onehot.astype(jnp.float32) @ bstart.astype(jnp.float32)).astype(jnp.int32)
    p = bstart_g * BM + rank

    vals = jnp.stack([t_flat.astype(jnp.float32), tw.reshape(-1)], axis=-1)
    packed = (
        jnp.zeros((NUM_BLOCKS * BM, 2), jnp.float32)
        .at[:, 0].set(float(SEQ))
        .at[p].set(vals)
    )
    tok_pad = packed[:, 0].astype(jnp.int32)
    w_pad = packed[:, 1]

    blk_ids = jnp.arange(NUM_BLOCKS, dtype=jnp.int32)
    q = jnp.minimum(blk_ids, n_active - 1)
    be = jnp.sum(
        (bcum[None, :] <= q[:, None]).astype(jnp.int32), axis=1
    ).astype(jnp.int32)
    block_expert = jnp.minimum(be, NUM_EXPERTS - 1)
    active = (blk_ids < n_active).astype(jnp.int32)

    grid_spec = pltpu.PrefetchScalarGridSpec(
        num_scalar_prefetch=2,
        grid=(NUM_BLOCKS,),
        in_specs=[
            pl.BlockSpec((SEQ, D_MODEL), lambda i, be_r, a_r: (0, 0)),
            pl.BlockSpec((BM, 1), lambda i, be_r, a_r: (i, 0)),
            pl.BlockSpec((1, 8, BM), lambda i, be_r, a_r: (i, 0, 0)),
            pl.BlockSpec((1, 8, BM), lambda i, be_r, a_r: (i, 0, 0)),
            pl.BlockSpec((1, D_MODEL, D_FF), lambda i, be_r, a_r: (be_r[i], 0, 0)),
            pl.BlockSpec((1, D_MODEL, D_FF), lambda i, be_r, a_r: (be_r[i], 0, 0)),
            pl.BlockSpec((1, D_FF, D_MODEL), lambda i, be_r, a_r: (be_r[i], 0, 0)),
        ],
        out_specs=pl.BlockSpec((SEQ, D_MODEL), lambda i, be_r, a_r: (0, 0)),
    )

    out = pl.pallas_call(
        _moe_block_kernel,
        grid_spec=grid_spec,
        out_shape=jax.ShapeDtypeStruct((SEQ, D_MODEL), jnp.float32),
    )(
        block_expert,
        active,
        x0,
        tok_pad.reshape(NUM_BLOCKS * BM, 1),
        jnp.broadcast_to(tok_pad.reshape(NUM_BLOCKS, 1, BM), (NUM_BLOCKS, 8, BM)),
        jnp.broadcast_to(w_pad.reshape(NUM_BLOCKS, 1, BM), (NUM_BLOCKS, 8, BM)),
        Wg,
        Wu,
        Wd,
    )
    return out[None]
